# baseline (speedup 1.0000x reference)
import functools

import jax
import jax.numpy as jnp
from jax import lax
from jax.experimental import pallas as pl
from jax.experimental.pallas import tpu as pltpu
from jax.experimental.pallas import tpu_sc as plsc

NC = 2
NS = 16
NW = NC * NS
LANES = 16

D = 64
SUB = 128
CHUNK = 256
SUBS = CHUNK // SUB
NBUF = 2
EPS = 1e-5
UNROLL = 4


def _rsqrt(x):
    i = lax.bitcast_convert_type(x, jnp.int32)
    i = jnp.full_like(i, 0x5F3759DF) - lax.shift_right_arithmetic(i, jnp.full_like(i, 1))
    y = lax.bitcast_convert_type(i, jnp.float32)
    h = x * jnp.float32(0.5)
    for _ in range(2):
        y = y * (jnp.float32(1.5) - h * y * y)
    return y


def _lane_sum(v, perms):
    for p in perms:
        v = v + jnp.take_along_axis(v, p, axis=0)
    return v


def _body(idx_hbm, table_hbm, gamma_hbm, beta_hbm, out_hbm,
          idx_v, rows_v, outs_v, gam_v, bet_v, gsems, ssems):
    wid = lax.axis_index("s") * NC + lax.axis_index("c")
    total_rows = out_hbm.shape[0]
    rows_per_w = total_rows // NW
    nchunks = rows_per_w // CHUNK
    idx_rows_base = wid * (rows_per_w // SUB)
    row_base = wid * rows_per_w

    pltpu.sync_copy(gamma_hbm, gam_v)
    pltpu.sync_copy(beta_hbm, bet_v)
    gs = [gam_v[pl.ds(k * LANES, LANES)] for k in range(4)]
    bs = [bet_v[pl.ds(k * LANES, LANES)] for k in range(4)]
    iota = lax.iota(jnp.int32, LANES)
    perms = [lax.bitwise_xor(iota, jnp.full_like(iota, s)) for s in (1, 2, 4, 8)]

    def fire_gather(g, b):
        pltpu.sync_copy(idx_hbm.at[pl.ds(idx_rows_base + g * SUBS, SUBS)],
                        idx_v[b])
        for j in range(SUBS):
            pltpu.async_copy(table_hbm.at[idx_v[b].at[j]],
                             rows_v[b].at[pl.ds(j * SUB, SUB)], gsems[b])

    def wait_gather(b):
        for j in range(SUBS):
            pltpu.make_async_copy(table_hbm.at[idx_v[b].at[j]],
                                  rows_v[b].at[pl.ds(j * SUB, SUB)],
                                  gsems[b]).wait()

    def fire_store(g, b):
        pltpu.async_copy(outs_v[b], out_hbm.at[pl.ds(row_base + g * CHUNK, CHUNK)],
                         ssems[b])

    def wait_store(g, b):
        pltpu.make_async_copy(outs_v[b], out_hbm.at[pl.ds(row_base + g * CHUNK, CHUNK)],
                              ssems[b]).wait()

    def compute(b):
        rv = rows_v[b]
        ov = outs_v[b]

        def _row(r, carry):
            xs = [rv[r, pl.ds(k * LANES, LANES)] for k in range(4)]
            s = (xs[0] + xs[1]) + (xs[2] + xs[3])
            q = (xs[0] * xs[0] + xs[1] * xs[1]) + (xs[2] * xs[2] + xs[3] * xs[3])
            ssum = _lane_sum(s, perms)
            qsum = _lane_sum(q, perms)
            mean = ssum * jnp.float32(1.0 / D)
            var = qsum * jnp.float32(1.0 / D) - mean * mean
            a = _rsqrt(var + jnp.float32(EPS))
            b_ = -mean * a
            for k in range(4):
                ov[r, pl.ds(k * LANES, LANES)] = (xs[k] * a + b_) * gs[k] + bs[k]
            return carry

        lax.fori_loop(0, CHUNK, _row, 0, unroll=UNROLL)

    fire_gather(0, 0)
    fire_gather(1, 1)

    def steady(g, b):
        wait_gather(b)
        compute(b)
        fire_store(g, b)
        wait_store(g, b)
        fire_gather(g + 2, b)

    def pair_body(p, carry):
        g = p * NBUF
        steady(g, 0)
        steady(g + 1, 1)
        return carry

    lax.fori_loop(0, nchunks // NBUF - 1, pair_body, 0)

    for g, b in ((nchunks - 2, 0), (nchunks - 1, 1)):
        wait_gather(b)
        compute(b)
        fire_store(g, b)
        wait_store(g, b)


@jax.jit
def _run(idx2d, table, gamma, beta):
    total_rows = idx2d.shape[0] * idx2d.shape[1]
    mesh = plsc.VectorSubcoreMesh(core_axis_name="c", subcore_axis_name="s")
    kern = pl.kernel(
        _body,
        out_type=jax.ShapeDtypeStruct((total_rows, D), jnp.float32),
        mesh=mesh,
        scratch_types=[
            [pltpu.VMEM((SUBS, SUB), jnp.int32) for _ in range(NBUF)],
            [pltpu.VMEM((CHUNK, D), jnp.float32) for _ in range(NBUF)],
            [pltpu.VMEM((CHUNK, D), jnp.float32) for _ in range(NBUF)],
            pltpu.VMEM((D,), jnp.float32),
            pltpu.VMEM((D,), jnp.float32),
            [pltpu.SemaphoreType.DMA for _ in range(NBUF)],
            [pltpu.SemaphoreType.DMA for _ in range(NBUF)],
        ],
        compiler_params=pltpu.CompilerParams(use_tc_tiling_on_sc=False),
    )
    return kern(idx2d, table, gamma, beta)


def kernel(shell_indices, table, gamma, beta):
    b, h = shell_indices.shape
    idx2d = shell_indices.astype(jnp.int32).reshape(-1).reshape(-1, SUB)
    out = _run(idx2d, table, gamma, beta)
    return out.reshape(b, h, D)


# speedup vs baseline: 1.4697x; 1.4697x over previous
import jax
import jax.numpy as jnp
from jax import lax
from jax.experimental import pallas as pl
from jax.experimental.pallas import tpu as pltpu
from jax.experimental.pallas import tpu_sc as plsc

NC = 2
NS = 16
NW = NC * NS
LANES = 16

D = 64
SUB = 128
CHUNK = 512
SUBS = CHUNK // SUB
NBUF = 2
EPS = 1e-5
UNROLL = 4


def _rsqrt(x):
    i = lax.bitcast_convert_type(x, jnp.int32)
    i = jnp.full_like(i, 0x5F3759DF) - lax.shift_right_arithmetic(i, jnp.full_like(i, 1))
    y = lax.bitcast_convert_type(i, jnp.float32)
    h = x * jnp.float32(0.5)
    for _ in range(2):
        y = y * (jnp.float32(1.5) - h * y * y)
    return y


def _lane_sum(v, perms):
    for p in perms:
        v = v + jnp.take_along_axis(v, p, axis=0)
    return v


def _body(idx_hbm, table_hbm, gamma_hbm, beta_hbm, out_hbm,
          idx_v, rows_v, gam_v, bet_v, gsems, ssems):
    wid = lax.axis_index("s") * NC + lax.axis_index("c")
    total_rows = out_hbm.shape[0]
    rows_per_w = total_rows // NW
    nchunks = rows_per_w // CHUNK
    idx_rows = rows_per_w // SUB
    row_base = wid * rows_per_w

    pltpu.sync_copy(idx_hbm.at[pl.ds(wid * idx_rows, idx_rows)], idx_v)
    pltpu.sync_copy(gamma_hbm, gam_v)
    pltpu.sync_copy(beta_hbm, bet_v)
    gs = [gam_v[pl.ds(k * LANES, LANES)] for k in range(4)]
    bs = [bet_v[pl.ds(k * LANES, LANES)] for k in range(4)]
    iota = lax.iota(jnp.int32, LANES)
    perms = [lax.bitwise_xor(iota, jnp.full_like(iota, s)) for s in (1, 2, 4, 8)]

    def fire_gather(g, b):
        for j in range(SUBS):
            pltpu.async_copy(table_hbm.at[idx_v.at[g * SUBS + j]],
                             rows_v[b].at[pl.ds(j * SUB, SUB)], gsems[b])

    def wait_gather(g, b):
        for j in range(SUBS):
            pltpu.make_async_copy(table_hbm.at[idx_v.at[g * SUBS + j]],
                                  rows_v[b].at[pl.ds(j * SUB, SUB)],
                                  gsems[b]).wait()

    def fire_store(g, b):
        pltpu.async_copy(rows_v[b], out_hbm.at[pl.ds(row_base + g * CHUNK, CHUNK)],
                         ssems[b])

    def wait_store(g, b):
        pltpu.make_async_copy(rows_v[b], out_hbm.at[pl.ds(row_base + g * CHUNK, CHUNK)],
                              ssems[b]).wait()

    def compute(b):
        rv = rows_v[b]

        def _row(r, carry):
            xs = [rv[r, pl.ds(k * LANES, LANES)] for k in range(4)]
            s = (xs[0] + xs[1]) + (xs[2] + xs[3])
            q = (xs[0] * xs[0] + xs[1] * xs[1]) + (xs[2] * xs[2] + xs[3] * xs[3])
            ssum = _lane_sum(s, perms)
            qsum = _lane_sum(q, perms)
            mean = ssum * jnp.float32(1.0 / D)
            var = qsum * jnp.float32(1.0 / D) - mean * mean
            a = _rsqrt(var + jnp.float32(EPS))
            b_ = -mean * a
            for k in range(4):
                rv[r, pl.ds(k * LANES, LANES)] = (xs[k] * a + b_) * gs[k] + bs[k]
            return carry

        lax.fori_loop(0, CHUNK, _row, 0, unroll=UNROLL)

    fire_gather(0, 0)
    fire_gather(1, 1)

    def steady(g, b):
        wait_gather(g, b)
        compute(b)
        fire_store(g, b)
        wait_store(g, b)
        fire_gather(g + 2, b)

    def pair_body(p, carry):
        g = p * NBUF
        steady(g, 0)
        steady(g + 1, 1)
        return carry

    lax.fori_loop(0, nchunks // NBUF - 1, pair_body, 0)

    for g, b in ((nchunks - 2, 0), (nchunks - 1, 1)):
        wait_gather(g, b)
        compute(b)
        fire_store(g, b)
        wait_store(g, b)


@jax.jit
def _run(idx2d, table, gamma, beta):
    total_rows = idx2d.shape[0] * idx2d.shape[1]
    idx_rows_per_w = total_rows // NW // SUB
    mesh = plsc.VectorSubcoreMesh(core_axis_name="c", subcore_axis_name="s")
    kern = pl.kernel(
        _body,
        out_type=jax.ShapeDtypeStruct((total_rows, D), jnp.float32),
        mesh=mesh,
        scratch_types=[
            pltpu.VMEM((idx_rows_per_w, SUB), jnp.int32),
            [pltpu.VMEM((CHUNK, D), jnp.float32) for _ in range(NBUF)],
            pltpu.VMEM((D,), jnp.float32),
            pltpu.VMEM((D,), jnp.float32),
            [pltpu.SemaphoreType.DMA for _ in range(NBUF)],
            [pltpu.SemaphoreType.DMA for _ in range(NBUF)],
        ],
        compiler_params=pltpu.CompilerParams(use_tc_tiling_on_sc=False),
    )
    return kern(idx2d, table, gamma, beta)


def kernel(shell_indices, table, gamma, beta):
    b, h = shell_indices.shape
    idx2d = shell_indices.astype(jnp.int32).reshape(-1).reshape(-1, SUB)
    out = _run(idx2d, table, gamma, beta)
    return out.reshape(b, h, D)
